# baseline (speedup 1.0000x reference)
import functools

import jax
import jax.numpy as jnp
from jax.experimental import pallas as pl
from jax.experimental.pallas import tpu as pltpu

_B, _D, _H, _W = 8, 256, 32, 32


def _pos_body(colv, rowv, out_ref, slab, sem):
    hh = _H // 8
    cols = colv[...]
    rows = rowv[...]
    copies = []
    for h in range(8):
        sl = pl.ds(h * hh, hh)
        slab[sl, :, 0:_D] = jnp.broadcast_to(cols[None], (hh, _W, _D))
        slab[sl, :, _D : 2 * _D] = jnp.broadcast_to(
            rows[h * hh : (h + 1) * hh][:, None, :], (hh, _W, _D)
        )
        for b in range(_B):
            c = pltpu.make_async_copy(slab.at[sl], out_ref.at[b, sl], sem)
            c.start()
            copies.append(c)
    for c in copies:
        c.wait()


_pos_call = functools.partial(
    pl.pallas_call,
    grid=(1,),
    in_specs=[
        pl.BlockSpec((_W, _D), lambda g: (0, 0)),
        pl.BlockSpec((_H, _D), lambda g: (0, 0)),
    ],
    out_specs=pl.BlockSpec(memory_space=pltpu.MemorySpace.HBM),
    out_shape=jax.ShapeDtypeStruct((_B, _H, _W, 2 * _D), jnp.float32),
    scratch_shapes=[
        pltpu.VMEM((_H, _W, 2 * _D), jnp.float32),
        pltpu.SemaphoreType.DMA,
    ],
)(_pos_body)


def kernel(x, row_embed, col_embed):
    del x
    out = _pos_call(col_embed, row_embed)
    return out.transpose((0, 3, 1, 2))
